# Initial kernel scaffold; baseline (speedup 1.0000x reference)
import jax
import jax.numpy as jnp
from jax.experimental import pallas as pl


def kernel(signals, neighbor_actions, W1, att_src1, att_dst1, b1, W2, att_src2, att_dst2, b2, in_w, in_b, out_w, out_b, lm_w, lm_b, ap_w, ap_b):
    raise NotImplementedError("write your pallas kernel here")


# speedup vs baseline: 1028.6926x; 1028.6926x over previous
import functools

import jax
import jax.numpy as jnp
from jax import lax
from jax.experimental import pallas as pl

NUM_AGENTS = 32
ACTION_DIM = 8
NUM_BELIEF = 120
HIDDEN = 32
HEADS = 4
FEAT = HIDDEN * HEADS


def _gat_block(h3, asf, adf, bias, nb):
    A, H, C, F = NUM_AGENTS, HEADS, HIDDEN, FEAT
    hflat = h3.reshape(nb * A, F)
    gi = lax.broadcasted_iota(jnp.int32, (F, H), 0) // C
    gj = lax.broadcasted_iota(jnp.int32, (F, H), 1)
    G = (gi == gj).astype(jnp.float32)
    a_src = jnp.dot(hflat * asf, G, preferred_element_type=jnp.float32)
    a_dst = jnp.dot(hflat * adf, G, preferred_element_type=jnp.float32)
    a_src3 = a_src.reshape(nb, A, H)
    a_dst3 = a_dst.reshape(nb, A, H)
    asrcT = jnp.swapaxes(a_src3, 1, 2)

    i_io = lax.broadcasted_iota(jnp.int32, (nb, A, A), 2)
    j_io = lax.broadcasted_iota(jnp.int32, (nb, A, A), 1)
    diag = i_io == j_io

    outs = []
    for hd in range(H):
        sT = asrcT[:, hd, :][:, None, :]
        dc = a_dst3[:, :, hd][:, :, None]
        L = sT + dc
        L = jnp.where(L >= 0, L, 0.2 * L)
        E = jnp.where(diag, 0.0, jnp.exp(L))
        den = jnp.sum(E, axis=2, keepdims=True)
        Wn = E / (den + 1e-16)
        hv = h3[:, :, hd * C:(hd + 1) * C]
        o = lax.dot_general(Wn, hv, (((2,), (1,)), ((0,), (0,))),
                            preferred_element_type=jnp.float32)
        outs.append(o)
    out = jnp.concatenate(outs, axis=2) + bias[None, :, :]
    return jnp.maximum(out, 0.0)


def _tgnn_kernel(sig_ref, na_ref, w1a_ref, w1s_ref, as1_ref, ad1_ref, b1_ref,
                 w2_ref, as2_ref, ad2_ref, b2_ref,
                 vw_ref, vb_ref, ow_ref, ob_ref, lw_ref, lb_ref, aw_ref, ab_ref,
                 out_ref):
    nb = sig_ref.shape[0]
    A, F = NUM_AGENTS, FEAT
    acts = na_ref[...].reshape(nb * A, ACTION_DIM)
    h = jnp.dot(acts, w1a_ref[...], preferred_element_type=jnp.float32)
    hsig = jnp.dot(sig_ref[...], w1s_ref[...], preferred_element_type=jnp.float32)
    h3 = h.reshape(nb, A, F)
    kmask = lax.broadcasted_iota(jnp.int32, (nb, A, F), 1) == 0
    h3 = h3 + jnp.where(kmask, hsig[:, None, :], 0.0)

    x = _gat_block(h3, as1_ref[...], ad1_ref[...], b1_ref[...], nb)
    h2 = jnp.dot(x.reshape(nb * A, F), w2_ref[...],
                 preferred_element_type=jnp.float32).reshape(nb, A, F)
    x2 = _gat_block(h2, as2_ref[...], ad2_ref[...], b2_ref[...], nb)

    ego = x2[:, 0, :]
    v = jnp.dot(ego, vw_ref[...], preferred_element_type=jnp.float32) + vb_ref[...]
    f = jnp.dot(v, ow_ref[...], preferred_element_type=jnp.float32) + ob_ref[...]
    z = jnp.dot(f, lw_ref[...], preferred_element_type=jnp.float32) + lb_ref[...]
    out_ref[...] = jnp.dot(z, aw_ref[...], preferred_element_type=jnp.float32) + ab_ref[...]


@functools.partial(jax.jit, static_argnames=())
def kernel(signals, neighbor_actions, W1, att_src1, att_dst1, b1, W2,
           att_src2, att_dst2, b2, in_w, in_b, out_w, out_b, lm_w, lm_b,
           ap_w, ap_b):
    B = signals.shape[0]
    NB = 64
    grid = (B // NB,)

    na3 = neighbor_actions.reshape(B, NUM_AGENTS, ACTION_DIM)
    w1aT = W1[:, NUM_BELIEF:].T
    w1sT = W1[:, :NUM_BELIEF].T
    w2T = W2.T
    vwT = in_w[2 * FEAT:].T
    vb = in_b[2 * FEAT:].reshape(1, -1)
    owT = out_w.T
    lwT = lm_w.T
    awT = ap_w.T

    def r2(v):
        return v.reshape(1, -1)

    out = pl.pallas_call(
        _tgnn_kernel,
        grid=grid,
        in_specs=[
            pl.BlockSpec((NB, NUM_BELIEF), lambda i: (i, 0)),
            pl.BlockSpec((NB, NUM_AGENTS, ACTION_DIM), lambda i: (i, 0, 0)),
            pl.BlockSpec(w1aT.shape, lambda i: (0, 0)),
            pl.BlockSpec(w1sT.shape, lambda i: (0, 0)),
            pl.BlockSpec((1, FEAT), lambda i: (0, 0)),
            pl.BlockSpec((1, FEAT), lambda i: (0, 0)),
            pl.BlockSpec((1, FEAT), lambda i: (0, 0)),
            pl.BlockSpec(w2T.shape, lambda i: (0, 0)),
            pl.BlockSpec((1, FEAT), lambda i: (0, 0)),
            pl.BlockSpec((1, FEAT), lambda i: (0, 0)),
            pl.BlockSpec((1, FEAT), lambda i: (0, 0)),
            pl.BlockSpec(vwT.shape, lambda i: (0, 0)),
            pl.BlockSpec((1, FEAT), lambda i: (0, 0)),
            pl.BlockSpec(owT.shape, lambda i: (0, 0)),
            pl.BlockSpec((1, FEAT), lambda i: (0, 0)),
            pl.BlockSpec(lwT.shape, lambda i: (0, 0)),
            pl.BlockSpec((1, lwT.shape[1]), lambda i: (0, 0)),
            pl.BlockSpec(awT.shape, lambda i: (0, 0)),
            pl.BlockSpec((1, awT.shape[1]), lambda i: (0, 0)),
        ],
        out_specs=pl.BlockSpec((NB, ACTION_DIM * NUM_AGENTS), lambda i: (i, 0)),
        out_shape=jax.ShapeDtypeStruct((B, ACTION_DIM * NUM_AGENTS), jnp.float32),
    )(signals, na3, w1aT, w1sT,
      att_src1.reshape(1, -1), att_dst1.reshape(1, -1), r2(b1),
      w2T, att_src2.reshape(1, -1), att_dst2.reshape(1, -1), r2(b2),
      vwT, vb, owT, r2(out_b), lwT, r2(lm_b), awT, r2(ap_b))
    return out


# speedup vs baseline: 2842.2441x; 2.7630x over previous
import functools

import jax
import jax.numpy as jnp
from jax import lax
from jax.experimental import pallas as pl

NUM_AGENTS = 32
ACTION_DIM = 8
NUM_BELIEF = 120
HIDDEN = 32
HEADS = 4
FEAT = HIDDEN * HEADS


def _gat_block(h3, asf, adf, bias, nb):
    A, H, C, F = NUM_AGENTS, HEADS, HIDDEN, FEAT
    hflat = h3.reshape(nb * A, F)
    bi = lax.broadcasted_iota(jnp.int32, (F, F), 0) // C
    bj = lax.broadcasted_iota(jnp.int32, (F, F), 1) // C
    BD = (bi == bj).astype(jnp.float32)
    a_src_bc = jnp.dot(hflat * asf, BD, preferred_element_type=jnp.float32)
    a_dst_bc = jnp.dot(hflat * adf, BD, preferred_element_type=jnp.float32)
    sel = (lax.broadcasted_iota(jnp.int32, (A, F), 0)
           == lax.broadcasted_iota(jnp.int32, (A, F), 1) % C).astype(jnp.float32)
    asrcT = jnp.sum(a_src_bc.reshape(nb, A, F) * sel[None], axis=1)
    L = asrcT[:, None, :] + a_dst_bc.reshape(nb, A, F)
    L = jnp.where(L >= 0, L, 0.2 * L)
    j_io = lax.broadcasted_iota(jnp.int32, (nb, A, F), 1)
    i_io = lax.broadcasted_iota(jnp.int32, (nb, A, F), 2) % C
    E = jnp.where(j_io == i_io, 0.0, jnp.exp(L))
    den = jnp.dot(E.reshape(nb * A, F), BD,
                  preferred_element_type=jnp.float32).reshape(nb, A, F)
    Wn = E / (den + 1e-16)
    Hbig = jnp.concatenate([h3, h3, h3, h3], axis=1)
    ri = lax.broadcasted_iota(jnp.int32, (nb, F, F), 1) // C
    ci = lax.broadcasted_iota(jnp.int32, (nb, F, F), 2) // C
    Hbig = jnp.where(ri == ci, Hbig, 0.0)
    out = lax.dot_general(Wn, Hbig, (((2,), (1,)), ((0,), (0,))),
                          preferred_element_type=jnp.float32)
    out = out + bias[None, :, :]
    return jnp.maximum(out, 0.0)


def _tgnn_kernel(sig_ref, na_ref, w1a_ref, w1s_ref, as1_ref, ad1_ref, b1_ref,
                 w2_ref, as2_ref, ad2_ref, b2_ref,
                 vw_ref, vb_ref, ow_ref, ob_ref, lw_ref, lb_ref, aw_ref, ab_ref,
                 out_ref):
    nb = sig_ref.shape[0]
    A, F = NUM_AGENTS, FEAT
    acts = na_ref[...].reshape(nb * A, ACTION_DIM)
    h = jnp.dot(acts, w1a_ref[...], preferred_element_type=jnp.float32)
    hsig = jnp.dot(sig_ref[...], w1s_ref[...], preferred_element_type=jnp.float32)
    h3 = h.reshape(nb, A, F)
    kmask = lax.broadcasted_iota(jnp.int32, (nb, A, F), 1) == 0
    h3 = h3 + jnp.where(kmask, hsig[:, None, :], 0.0)

    x = _gat_block(h3, as1_ref[...], ad1_ref[...], b1_ref[...], nb)
    h2 = jnp.dot(x.reshape(nb * A, F), w2_ref[...],
                 preferred_element_type=jnp.float32).reshape(nb, A, F)
    x2 = _gat_block(h2, as2_ref[...], ad2_ref[...], b2_ref[...], nb)

    ego = x2[:, 0, :]
    v = jnp.dot(ego, vw_ref[...], preferred_element_type=jnp.float32) + vb_ref[...]
    f = jnp.dot(v, ow_ref[...], preferred_element_type=jnp.float32) + ob_ref[...]
    z = jnp.dot(f, lw_ref[...], preferred_element_type=jnp.float32) + lb_ref[...]
    out_ref[...] = jnp.dot(z, aw_ref[...], preferred_element_type=jnp.float32) + ab_ref[...]


@functools.partial(jax.jit, static_argnames=())
def kernel(signals, neighbor_actions, W1, att_src1, att_dst1, b1, W2,
           att_src2, att_dst2, b2, in_w, in_b, out_w, out_b, lm_w, lm_b,
           ap_w, ap_b):
    B = signals.shape[0]
    NB = 64
    grid = (B // NB,)

    na3 = neighbor_actions.reshape(B, NUM_AGENTS, ACTION_DIM)
    w1aT = W1[:, NUM_BELIEF:].T
    w1sT = W1[:, :NUM_BELIEF].T
    w2T = W2.T
    vwT = in_w[2 * FEAT:].T
    vb = in_b[2 * FEAT:].reshape(1, -1)
    owT = out_w.T
    lwT = lm_w.T
    awT = ap_w.T

    def r2(v):
        return v.reshape(1, -1)

    out = pl.pallas_call(
        _tgnn_kernel,
        grid=grid,
        in_specs=[
            pl.BlockSpec((NB, NUM_BELIEF), lambda i: (i, 0)),
            pl.BlockSpec((NB, NUM_AGENTS, ACTION_DIM), lambda i: (i, 0, 0)),
            pl.BlockSpec(w1aT.shape, lambda i: (0, 0)),
            pl.BlockSpec(w1sT.shape, lambda i: (0, 0)),
            pl.BlockSpec((1, FEAT), lambda i: (0, 0)),
            pl.BlockSpec((1, FEAT), lambda i: (0, 0)),
            pl.BlockSpec((1, FEAT), lambda i: (0, 0)),
            pl.BlockSpec(w2T.shape, lambda i: (0, 0)),
            pl.BlockSpec((1, FEAT), lambda i: (0, 0)),
            pl.BlockSpec((1, FEAT), lambda i: (0, 0)),
            pl.BlockSpec((1, FEAT), lambda i: (0, 0)),
            pl.BlockSpec(vwT.shape, lambda i: (0, 0)),
            pl.BlockSpec((1, FEAT), lambda i: (0, 0)),
            pl.BlockSpec(owT.shape, lambda i: (0, 0)),
            pl.BlockSpec((1, FEAT), lambda i: (0, 0)),
            pl.BlockSpec(lwT.shape, lambda i: (0, 0)),
            pl.BlockSpec((1, lwT.shape[1]), lambda i: (0, 0)),
            pl.BlockSpec(awT.shape, lambda i: (0, 0)),
            pl.BlockSpec((1, awT.shape[1]), lambda i: (0, 0)),
        ],
        out_specs=pl.BlockSpec((NB, ACTION_DIM * NUM_AGENTS), lambda i: (i, 0)),
        out_shape=jax.ShapeDtypeStruct((B, ACTION_DIM * NUM_AGENTS), jnp.float32),
    )(signals, na3, w1aT, w1sT,
      att_src1.reshape(1, -1), att_dst1.reshape(1, -1), r2(b1),
      w2T, att_src2.reshape(1, -1), att_dst2.reshape(1, -1), r2(b2),
      vwT, vb, owT, r2(out_b), lwT, r2(lm_b), awT, r2(ap_b))
    return out


# speedup vs baseline: 3074.3061x; 1.0816x over previous
import jax
import jax.numpy as jnp
from jax import lax
from jax.experimental import pallas as pl

NUM_AGENTS = 32
ACTION_DIM = 8
NUM_BELIEF = 120
HIDDEN = 32
HEADS = 4
FEAT = HIDDEN * HEADS

_DN_T = (((1,), (1,)), ((), ()))


def _dot_t(x, w):
    return lax.dot_general(x, w, _DN_T, preferred_element_type=jnp.float32)


def _gat_block(h3, asf, adf, bias, nb):
    A, C, F = NUM_AGENTS, HIDDEN, FEAT
    hflat = h3.reshape(nb * A, F)
    bi = lax.broadcasted_iota(jnp.int32, (F, F), 0) // C
    bj = lax.broadcasted_iota(jnp.int32, (F, F), 1) // C
    BD = (bi == bj).astype(jnp.float32)
    a_src_bc = jnp.dot(hflat * asf, BD, preferred_element_type=jnp.float32)
    a_dst_bc = jnp.dot(hflat * adf, BD, preferred_element_type=jnp.float32)
    sel = (lax.broadcasted_iota(jnp.int32, (A, F), 0)
           == lax.broadcasted_iota(jnp.int32, (A, F), 1) % C).astype(jnp.float32)
    asrcT = jnp.sum(a_src_bc.reshape(nb, A, F) * sel[None], axis=1)
    L = asrcT[:, None, :] + a_dst_bc.reshape(nb, A, F)
    L = jnp.where(L >= 0, L, 0.2 * L)
    j_io = lax.broadcasted_iota(jnp.int32, (nb, A, F), 1)
    i_io = lax.broadcasted_iota(jnp.int32, (nb, A, F), 2) % C
    E = jnp.where(j_io == i_io, 0.0, jnp.exp(L))
    den = jnp.dot(E.reshape(nb * A, F), BD,
                  preferred_element_type=jnp.float32).reshape(nb, A, F)
    Wn = E / (den + 1e-16)
    Hbig = jnp.concatenate([h3, h3, h3, h3], axis=1)
    ri = lax.broadcasted_iota(jnp.int32, (nb, F, F), 1) // C
    ci = lax.broadcasted_iota(jnp.int32, (nb, F, F), 2) // C
    Hbig = jnp.where(ri == ci, Hbig, 0.0)
    out = lax.dot_general(Wn, Hbig, (((2,), (1,)), ((0,), (0,))),
                          preferred_element_type=jnp.float32)
    out = out + bias[None, :, :]
    return jnp.maximum(out, 0.0)


def _tgnn_kernel(sig_ref, na_ref, w1_ref, as1_ref, ad1_ref, b1_ref,
                 w2_ref, as2_ref, ad2_ref, b2_ref,
                 vw_ref, vb_ref, ow_ref, ob_ref, lw_ref, lb_ref, aw_ref, ab_ref,
                 out_ref):
    nb = sig_ref.shape[0]
    A, F = NUM_AGENTS, FEAT
    acts = na_ref[...].reshape(nb * A, ACTION_DIM)
    w1 = w1_ref[...]
    h = _dot_t(acts, w1[:, NUM_BELIEF:])
    hsig = _dot_t(sig_ref[...], w1[:, :NUM_BELIEF])
    h3 = h.reshape(nb, A, F)
    kmask = lax.broadcasted_iota(jnp.int32, (nb, A, F), 1) == 0
    h3 = h3 + jnp.where(kmask, hsig[:, None, :], 0.0)

    x = _gat_block(h3, as1_ref[...], ad1_ref[...], b1_ref[...], nb)
    h2 = _dot_t(x.reshape(nb * A, F), w2_ref[...]).reshape(nb, A, F)
    x2 = _gat_block(h2, as2_ref[...], ad2_ref[...], b2_ref[...], nb)

    ego = x2[:, 0, :]
    v = _dot_t(ego, vw_ref[...]) + vb_ref[:, 2 * FEAT:]
    f = _dot_t(v, ow_ref[...]) + ob_ref[...]
    z = _dot_t(f, lw_ref[...]) + lb_ref[...]
    out_ref[...] = _dot_t(z, aw_ref[...]) + ab_ref[...]


@jax.jit
def kernel(signals, neighbor_actions, W1, att_src1, att_dst1, b1, W2,
           att_src2, att_dst2, b2, in_w, in_b, out_w, out_b, lm_w, lm_b,
           ap_w, ap_b):
    B = signals.shape[0]
    NB = 64
    grid = (B // NB,)

    na3 = neighbor_actions.reshape(B, NUM_AGENTS, ACTION_DIM)

    def full(a):
        nd = a.ndim
        return pl.BlockSpec(a.shape, lambda i, _n=nd: (0,) * _n)

    def r2(v):
        return v.reshape(1, -1)

    args = [
        signals, na3, W1,
        att_src1.reshape(1, -1), att_dst1.reshape(1, -1), r2(b1),
        W2, att_src2.reshape(1, -1), att_dst2.reshape(1, -1), r2(b2),
        in_w, r2(in_b), out_w, r2(out_b), lm_w, r2(lm_b), ap_w, r2(ap_b),
    ]
    in_specs = [
        pl.BlockSpec((NB, NUM_BELIEF), lambda i: (i, 0)),
        pl.BlockSpec((NB, NUM_AGENTS, ACTION_DIM), lambda i: (i, 0, 0)),
    ] + [full(a) for a in args[2:10]] + [
        pl.BlockSpec((FEAT, FEAT), lambda i: (2, 0)),
    ] + [full(a) for a in args[11:]]

    out = pl.pallas_call(
        _tgnn_kernel,
        grid=grid,
        in_specs=in_specs,
        out_specs=pl.BlockSpec((NB, ACTION_DIM * NUM_AGENTS), lambda i: (i, 0)),
        out_shape=jax.ShapeDtypeStruct((B, ACTION_DIM * NUM_AGENTS), jnp.float32),
    )(*args)
    return out


# speedup vs baseline: 3649.0656x; 1.1870x over previous
import jax
import jax.numpy as jnp
from jax import lax
from jax.experimental import pallas as pl

NUM_AGENTS = 32
ACTION_DIM = 8
NUM_BELIEF = 120
HIDDEN = 32
HEADS = 4
FEAT = HIDDEN * HEADS

_DN_T = (((1,), (1,)), ((), ()))


def _dot_t(x, w):
    return lax.dot_general(x, w, _DN_T, preferred_element_type=jnp.float32)


def _gat_block(h3, asf, adf, bias, nb):
    A, C, F = NUM_AGENTS, HIDDEN, FEAT
    hflat = h3.reshape(nb * A, F)
    bi = lax.broadcasted_iota(jnp.int32, (F, F), 0) // C
    bj = lax.broadcasted_iota(jnp.int32, (F, F), 1) // C
    BD = (bi == bj).astype(jnp.float32)
    a_src_bc = jnp.dot(hflat * asf, BD, preferred_element_type=jnp.float32)
    a_dst_bc = jnp.dot(hflat * adf, BD, preferred_element_type=jnp.float32)
    sel = (lax.broadcasted_iota(jnp.int32, (A, F), 0)
           == lax.broadcasted_iota(jnp.int32, (A, F), 1) % C).astype(jnp.float32)
    asrcT = jnp.sum(a_src_bc.reshape(nb, A, F) * sel[None], axis=1)
    L = asrcT[:, None, :] + a_dst_bc.reshape(nb, A, F)
    L = jnp.where(L >= 0, L, 0.2 * L)
    j_io = lax.broadcasted_iota(jnp.int32, (nb, A, F), 1)
    i_io = lax.broadcasted_iota(jnp.int32, (nb, A, F), 2) % C
    E = jnp.where(j_io == i_io, 0.0, jnp.exp(L))
    den = jnp.dot(E.reshape(nb * A, F), BD,
                  preferred_element_type=jnp.float32).reshape(nb, A, F)
    Wn = E / (den + 1e-16)
    Hbig = jnp.concatenate([h3, h3, h3, h3], axis=1)
    ri = lax.broadcasted_iota(jnp.int32, (nb, F, F), 1) // C
    ci = lax.broadcasted_iota(jnp.int32, (nb, F, F), 2) // C
    Hbig = jnp.where(ri == ci, Hbig, 0.0)
    out = lax.dot_general(Wn, Hbig, (((2,), (1,)), ((0,), (0,))),
                          preferred_element_type=jnp.float32)
    out = out + bias[None, :, :]
    return jnp.maximum(out, 0.0)


def _tgnn_kernel(sig_ref, na_ref, w1_ref, as1_ref, ad1_ref, b1_ref,
                 w2_ref, as2_ref, ad2_ref, b2_ref,
                 vw_ref, vb_ref, ow_ref, ob_ref, lw_ref, lb_ref, aw_ref, ab_ref,
                 out_ref):
    nb = sig_ref.shape[0]
    A, F = NUM_AGENTS, FEAT
    acts = na_ref[...].reshape(nb * A, ACTION_DIM)
    w1 = w1_ref[...]
    h = _dot_t(acts, w1[:, NUM_BELIEF:])
    hsig = _dot_t(sig_ref[...], w1[:, :NUM_BELIEF])
    h3 = h.reshape(nb, A, F)
    kmask = lax.broadcasted_iota(jnp.int32, (nb, A, F), 1) == 0
    h3 = h3 + jnp.where(kmask, hsig[:, None, :], 0.0)

    x = _gat_block(h3, as1_ref[...], ad1_ref[...], b1_ref[...], nb)
    h2 = _dot_t(x.reshape(nb * A, F), w2_ref[...]).reshape(nb, A, F)
    x2 = _gat_block(h2, as2_ref[...], ad2_ref[...], b2_ref[...], nb)

    ego = x2[:, 0, :]
    v = _dot_t(ego, vw_ref[...]) + vb_ref[:, 2 * FEAT:]
    f = _dot_t(v, ow_ref[...]) + ob_ref[...]
    z = _dot_t(f, lw_ref[...]) + lb_ref[...]
    out_ref[...] = _dot_t(z, aw_ref[...]) + ab_ref[...]


@jax.jit
def kernel(signals, neighbor_actions, W1, att_src1, att_dst1, b1, W2,
           att_src2, att_dst2, b2, in_w, in_b, out_w, out_b, lm_w, lm_b,
           ap_w, ap_b):
    B = signals.shape[0]
    NB = 256
    grid = (B // NB,)

    na3 = neighbor_actions.reshape(B, NUM_AGENTS, ACTION_DIM)

    def full(a):
        nd = a.ndim
        return pl.BlockSpec(a.shape, lambda i, _n=nd: (0,) * _n)

    def r2(v):
        return v.reshape(1, -1)

    args = [
        signals, na3, W1,
        att_src1.reshape(1, -1), att_dst1.reshape(1, -1), r2(b1),
        W2, att_src2.reshape(1, -1), att_dst2.reshape(1, -1), r2(b2),
        in_w, r2(in_b), out_w, r2(out_b), lm_w, r2(lm_b), ap_w, r2(ap_b),
    ]
    in_specs = [
        pl.BlockSpec((NB, NUM_BELIEF), lambda i: (i, 0)),
        pl.BlockSpec((NB, NUM_AGENTS, ACTION_DIM), lambda i: (i, 0, 0)),
    ] + [full(a) for a in args[2:10]] + [
        pl.BlockSpec((FEAT, FEAT), lambda i: (2, 0)),
    ] + [full(a) for a in args[11:]]

    out = pl.pallas_call(
        _tgnn_kernel,
        grid=grid,
        in_specs=in_specs,
        out_specs=pl.BlockSpec((NB, ACTION_DIM * NUM_AGENTS), lambda i: (i, 0)),
        out_shape=jax.ShapeDtypeStruct((B, ACTION_DIM * NUM_AGENTS), jnp.float32),
    )(*args)
    return out
